# Initial kernel scaffold; baseline (speedup 1.0000x reference)
import jax
import jax.numpy as jnp
from jax.experimental import pallas as pl


def kernel(features, centroids):
    raise NotImplementedError("write your pallas kernel here")


# speedup vs baseline: 7.4323x; 7.4323x over previous
import functools

import jax
import jax.numpy as jnp
from jax import lax
from jax.experimental import pallas as pl

N = 16384
D = 32
K = 512
BLOCK_N = 2048


def _assign_block(x_ref, c_ref, o_ref):
    x = x_ref[...]
    c = c_ref[...]
    s = jnp.dot(x, c, preferred_element_type=jnp.float32,
                precision=lax.Precision.HIGHEST)
    cn = jnp.sum(c * c, axis=0, keepdims=True)
    d = cn - 2.0 * s
    o_ref[...] = jnp.argmin(d, axis=-1).astype(jnp.float32)


@jax.jit
def kernel(features, centroids):
    grid = (N // BLOCK_N,)
    out = pl.pallas_call(
        _assign_block,
        grid=grid,
        in_specs=[
            pl.BlockSpec((BLOCK_N, D), lambda i: (i, 0)),
            pl.BlockSpec((D, K), lambda i: (0, 0)),
        ],
        out_specs=pl.BlockSpec((BLOCK_N,), lambda i: (i,)),
        out_shape=jax.ShapeDtypeStruct((N,), jnp.float32),
    )(features, centroids)
    return out[:, None]
